# baseline (speedup 1.0000x reference)
import functools

import jax
import jax.numpy as jnp
from jax import lax
from jax.experimental import pallas as pl
from jax.experimental.pallas import tpu as pltpu
from jax.experimental.pallas import tpu_sc as plsc

_N_FIELDS = 26
_VOCAB = 100000
_EMB = 32
_BATCH = 16384

_NC = 2
_NS = 16
_NW = _NC * _NS

_ROWS = _BATCH * _N_FIELDS
_ROWS_PER_W = _ROWS // _NW
_SUB = 128
_NSUB = 13
_CHUNK = _SUB * _NSUB
_NCHUNK = _ROWS_PER_W // _CHUNK


_cache = {}


def _build_gather_kernel():
    if "k" in _cache:
        return _cache["k"]
    mesh = plsc.VectorSubcoreMesh(core_axis_name="c", subcore_axis_name="s")

    @functools.partial(
        pl.kernel,
        mesh=mesh,
        out_type=jax.ShapeDtypeStruct((_ROWS, _EMB), jnp.float32),
        compiler_params=pltpu.CompilerParams(use_tc_tiling_on_sc=False),
        scratch_types=[
            pltpu.VMEM((_ROWS_PER_W // _SUB, _SUB), jnp.int32),
            pltpu.VMEM((_CHUNK, _EMB), jnp.float32),
            pltpu.SemaphoreType.DMA,
        ],
    )
    def _gather_kernel(idx_hbm, table_hbm, out_hbm, idx_v, rows_v, sem):
        wid = lax.axis_index("s") * _NC + lax.axis_index("c")
        nsub_w = _ROWS_PER_W // _SUB
        base_sub_w = wid * nsub_w
        pltpu.sync_copy(idx_hbm.at[pl.ds(base_sub_w, nsub_w)], idx_v)

        def chunk_body(ci, carry):
            base_sub = ci * _NSUB
            copies = []
            for j in range(_NSUB):
                copies.append(
                    pltpu.async_copy(
                        table_hbm.at[idx_v.at[base_sub + j]],
                        rows_v.at[pl.ds(j * _SUB, _SUB)],
                        sem,
                    )
                )
            for c in copies:
                c.wait()
            pltpu.sync_copy(
                rows_v,
                out_hbm.at[pl.ds((base_sub_w + base_sub) * _SUB, _CHUNK)],
            )
            return carry

        lax.fori_loop(0, _NCHUNK, chunk_body, 0)

    _cache["k"] = _gather_kernel
    return _gather_kernel


def kernel(numeric_features, categorical_features, mask, tables):
    cat = categorical_features.astype(jnp.int32)
    offs = (jnp.arange(_N_FIELDS, dtype=jnp.int32) * _VOCAB)[None, :]
    idx = (cat + offs).reshape(_ROWS // _SUB, _SUB)
    table2d = tables.reshape(_N_FIELDS * _VOCAB, _EMB)
    emb = _build_gather_kernel()(idx, table2d)
    scale = jnp.where(mask, 0.0, 1.0).astype(jnp.float32)
    embs_flat = emb.reshape(_BATCH, _N_FIELDS * _EMB) * scale
    return jnp.concatenate([embs_flat, numeric_features], axis=-1)
